# baseline (speedup 1.0000x reference)
import math

import jax
import jax.numpy as jnp
from jax import lax
from jax.experimental import pallas as pl
from jax.experimental.pallas import tpu as pltpu
from jax.experimental.pallas import tpu_sc as plsc

D_DIM = 256
BATCH = 128
SEQ = 1500
SEQ_PAD = 1536
CHUNK = 128
N_CHUNK = SEQ_PAD // CHUNK
TAIL = SEQ - (N_CHUNK - 1) * CHUNK
NW = 32
B_PER_W = BATCH // NW
SCALE = math.sqrt(D_DIM)
LANES = 16


def _pe_padded():
    position = jnp.arange(0.0, SEQ)[:, None]
    div_term = jnp.exp(jnp.arange(0.0, D_DIM, 2) * -(math.log(10000.0) / D_DIM))
    ang = position * div_term
    pe = jnp.zeros((SEQ, D_DIM), dtype=jnp.float32)
    pe = pe.at[:, 0::2].set(jnp.sin(ang))
    pe = pe.at[:, 1::2].set(jnp.cos(ang))
    return jnp.pad(pe, ((0, SEQ_PAD - SEQ), (0, 0)))


def _sc_body(x_hbm, table_hbm, pe_hbm, out_hbm, idx_v, pe_v, rows_v, gsem):
    wid = lax.axis_index("s") * 2 + lax.axis_index("c")
    b0 = wid * B_PER_W
    pltpu.sync_copy(x_hbm.at[pl.ds(b0, B_PER_W)], idx_v)

    def iter_body(i, carry):
        chunk = i // B_PER_W
        b_local = i % B_PER_W
        s_off = chunk * CHUNK

        @pl.when(b_local == 0)
        def _():
            pltpu.sync_copy(pe_hbm.at[pl.ds(s_off, CHUNK)], pe_v)

        idx_slice = idx_v.at[b_local, pl.ds(s_off, CHUNK)]
        pltpu.async_copy(table_hbm.at[idx_slice], rows_v, gsem).wait()

        def row_body(r, c2):
            for j in range(D_DIM // LANES):
                sl = pl.ds(j * LANES, LANES)
                rows_v[r, sl] = rows_v[r, sl] * SCALE + pe_v[r, sl]
            return c2

        lax.fori_loop(0, CHUNK, row_body, 0)

        b_g = b0 + b_local

        @pl.when(chunk < N_CHUNK - 1)
        def _():
            pltpu.sync_copy(rows_v, out_hbm.at[b_g, pl.ds(s_off, CHUNK)])

        @pl.when(chunk == N_CHUNK - 1)
        def _():
            pltpu.sync_copy(
                rows_v.at[pl.ds(0, TAIL)], out_hbm.at[b_g, pl.ds(s_off, TAIL)]
            )

        return carry

    lax.fori_loop(0, N_CHUNK * B_PER_W, iter_body, 0)


@jax.jit
def _impl(x, table):
    xp = jnp.pad(x, ((0, 0), (0, SEQ_PAD - SEQ)))
    pe = _pe_padded()
    mesh = plsc.VectorSubcoreMesh(core_axis_name="c", subcore_axis_name="s")
    k = pl.kernel(
        _sc_body,
        mesh=mesh,
        out_type=jax.ShapeDtypeStruct((BATCH, SEQ, D_DIM), jnp.float32),
        scratch_types=[
            pltpu.VMEM((B_PER_W, SEQ_PAD), jnp.int32),
            pltpu.VMEM((CHUNK, D_DIM), jnp.float32),
            pltpu.VMEM((CHUNK, D_DIM), jnp.float32),
            pltpu.SemaphoreType.DMA,
        ],
        compiler_params=pltpu.CompilerParams(use_tc_tiling_on_sc=False),
    )
    return k(xp, table, pe)


def kernel(x, table):
    return _impl(x, table)


# speedup vs baseline: 1.0472x; 1.0472x over previous
import math

import jax
import jax.numpy as jnp
from jax import lax
from jax.experimental import pallas as pl
from jax.experimental.pallas import tpu as pltpu
from jax.experimental.pallas import tpu_sc as plsc

D_DIM = 256
BATCH = 128
SEQ = 1500
SEQ_PAD = 1536
CHUNK = 128
N_CHUNK = SEQ_PAD // CHUNK
TAIL = SEQ - (N_CHUNK - 1) * CHUNK
NW = 32
B_PER_W = BATCH // NW
N_ITER = N_CHUNK * B_PER_W
SCALE = math.sqrt(D_DIM)
LANES = 16


def _pe_padded():
    position = jnp.arange(0.0, SEQ)[:, None]
    div_term = jnp.exp(jnp.arange(0.0, D_DIM, 2) * -(math.log(10000.0) / D_DIM))
    ang = position * div_term
    pe = jnp.stack([jnp.sin(ang), jnp.cos(ang)], axis=-1).reshape(SEQ, D_DIM)
    return jnp.pad(pe, ((0, SEQ_PAD - SEQ), (0, 0)))


def _sc_body(
    x_hbm, table_hbm, pe_hbm, out_hbm,
    idx_v, pe_v, rows0, rows1, gsem0, gsem1, wsem0, wsem1,
):
    wid = lax.axis_index("s") * 2 + lax.axis_index("c")
    b0 = wid * B_PER_W
    rows = [rows0, rows1]
    gsem = [gsem0, gsem1]
    wsem = [wsem0, wsem1]

    pltpu.sync_copy(x_hbm.at[pl.ds(b0, B_PER_W)], idx_v)

    def issue_gather(i, p):
        chunk = i // B_PER_W
        b_local = i % B_PER_W
        idx_slice = idx_v.at[b_local, pl.ds(chunk * CHUNK, CHUNK)]
        pltpu.async_copy(table_hbm.at[idx_slice], rows[p], gsem[p])

    def wait_gather(p):
        pltpu.make_async_copy(pe_hbm.at[pl.ds(0, CHUNK)], rows[p], gsem[p]).wait()

    def wait_write(p, chunk_prev):
        @pl.when(chunk_prev < N_CHUNK - 1)
        def _():
            pltpu.make_async_copy(
                rows[p], out_hbm.at[0, pl.ds(0, CHUNK)], wsem[p]
            ).wait()

        @pl.when(chunk_prev == N_CHUNK - 1)
        def _():
            pltpu.make_async_copy(
                rows[p].at[pl.ds(0, TAIL)], out_hbm.at[0, pl.ds(0, TAIL)], wsem[p]
            ).wait()

    issue_gather(0, 0)

    def loop_body(it, carry):
        for phase in range(2):
            i = it * 2 + phase
            p = phase
            q = 1 - phase
            chunk = i // B_PER_W
            b_local = i % B_PER_W
            s_off = chunk * CHUNK
            b_g = b0 + b_local

            @pl.when(i >= 1)
            def _():
                wait_write(q, (i - 1) // B_PER_W)

            @pl.when(i + 1 < N_ITER)
            def _():
                issue_gather(i + 1, q)

            @pl.when(b_local == 0)
            def _():
                pltpu.sync_copy(pe_hbm.at[pl.ds(s_off, CHUNK)], pe_v)

            wait_gather(p)

            def row_body(r, c2):
                for j in range(D_DIM // LANES):
                    sl = pl.ds(j * LANES, LANES)
                    rows[p][r, sl] = rows[p][r, sl] * SCALE + pe_v[r, sl]
                return c2

            lax.fori_loop(0, CHUNK, row_body, 0)

            @pl.when(chunk < N_CHUNK - 1)
            def _():
                pltpu.async_copy(rows[p], out_hbm.at[b_g, pl.ds(s_off, CHUNK)], wsem[p])

            @pl.when(chunk == N_CHUNK - 1)
            def _():
                pltpu.async_copy(
                    rows[p].at[pl.ds(0, TAIL)],
                    out_hbm.at[b_g, pl.ds(s_off, TAIL)],
                    wsem[p],
                )

        return carry

    lax.fori_loop(0, N_ITER // 2, loop_body, 0)

    wait_write(1, N_CHUNK - 1)


@jax.jit
def _impl(x, table):
    xp = jnp.pad(x, ((0, 0), (0, SEQ_PAD - SEQ)))
    pe = _pe_padded()
    mesh = plsc.VectorSubcoreMesh(core_axis_name="c", subcore_axis_name="s")
    k = pl.kernel(
        _sc_body,
        mesh=mesh,
        out_type=jax.ShapeDtypeStruct((BATCH, SEQ, D_DIM), jnp.float32),
        scratch_types=[
            pltpu.VMEM((B_PER_W, SEQ_PAD), jnp.int32),
            pltpu.VMEM((CHUNK, D_DIM), jnp.float32),
            pltpu.VMEM((CHUNK, D_DIM), jnp.float32),
            pltpu.VMEM((CHUNK, D_DIM), jnp.float32),
            pltpu.SemaphoreType.DMA,
            pltpu.SemaphoreType.DMA,
            pltpu.SemaphoreType.DMA,
            pltpu.SemaphoreType.DMA,
        ],
        compiler_params=pltpu.CompilerParams(use_tc_tiling_on_sc=False),
    )
    return k(xp, table, pe)


def kernel(x, table):
    return _impl(x, table)


# speedup vs baseline: 1.0746x; 1.0262x over previous
import math

import numpy as np
import jax
import jax.numpy as jnp
from jax import lax
from jax.experimental import pallas as pl
from jax.experimental.pallas import tpu as pltpu
from jax.experimental.pallas import tpu_sc as plsc

D_DIM = 256
BATCH = 128
SEQ = 1500
SEQ_PAD = 1536
CHUNK = 96
N_CHUNK = SEQ_PAD // CHUNK
TAIL = SEQ - (N_CHUNK - 1) * CHUNK
NW = 32
B_PER_W = BATCH // NW
N_ITER = N_CHUNK * B_PER_W
SCALE = math.sqrt(D_DIM)
LANES = 16


def _pe_padded_np():
    position = np.arange(0.0, SEQ, dtype=np.float64)[:, None]
    div_term = np.exp(
        np.arange(0.0, D_DIM, 2, dtype=np.float64) * -(math.log(10000.0) / D_DIM)
    )
    ang = position * div_term
    pe = np.zeros((SEQ_PAD, D_DIM), dtype=np.float32)
    pe[:SEQ, 0::2] = np.sin(ang)
    pe[:SEQ, 1::2] = np.cos(ang)
    return pe


_PE_CONST = _pe_padded_np()


def _sc_body(
    x_hbm, table_hbm, pe_hbm, out_hbm,
    idx_v, pe_v, rows0, rows1, rows2,
    gsem0, gsem1, gsem2, wsem0, wsem1, wsem2,
):
    wid = lax.axis_index("s") * 2 + lax.axis_index("c")
    b0 = wid * B_PER_W
    rows = [rows0, rows1, rows2]
    gsem = [gsem0, gsem1, gsem2]
    wsem = [wsem0, wsem1, wsem2]

    pltpu.sync_copy(x_hbm.at[pl.ds(b0, B_PER_W)], idx_v)

    def issue_gather(i, p):
        chunk = i // B_PER_W
        b_local = i % B_PER_W
        idx_slice = idx_v.at[b_local, pl.ds(chunk * CHUNK, CHUNK)]
        pltpu.async_copy(table_hbm.at[idx_slice], rows[p], gsem[p])

    def wait_gather(p):
        pltpu.make_async_copy(pe_hbm.at[pl.ds(0, CHUNK)], rows[p], gsem[p]).wait()

    def wait_write(p, chunk_prev):
        @pl.when(chunk_prev < N_CHUNK - 1)
        def _():
            pltpu.make_async_copy(
                rows[p], out_hbm.at[0, pl.ds(0, CHUNK)], wsem[p]
            ).wait()

        @pl.when(chunk_prev == N_CHUNK - 1)
        def _():
            pltpu.make_async_copy(
                rows[p].at[pl.ds(0, TAIL)], out_hbm.at[0, pl.ds(0, TAIL)], wsem[p]
            ).wait()

    def step(i, p, q):
        chunk = i // B_PER_W
        b_local = i % B_PER_W
        s_off = chunk * CHUNK
        b_g = b0 + b_local

        @pl.when(i >= 2)
        def _():
            wait_write(q, (i - 2) // B_PER_W)

        @pl.when(i + 1 < N_ITER)
        def _():
            issue_gather(i + 1, q)

        @pl.when(b_local == 0)
        def _():
            pltpu.sync_copy(pe_hbm.at[pl.ds(s_off, CHUNK)], pe_v)

        wait_gather(p)

        def row_body(r, c2):
            for j in range(D_DIM // LANES):
                sl = pl.ds(j * LANES, LANES)
                rows[p][r, sl] = rows[p][r, sl] * SCALE + pe_v[r, sl]
            return c2

        lax.fori_loop(0, CHUNK, row_body, 0)

        @pl.when(chunk < N_CHUNK - 1)
        def _():
            pltpu.async_copy(rows[p], out_hbm.at[b_g, pl.ds(s_off, CHUNK)], wsem[p])

        @pl.when(chunk == N_CHUNK - 1)
        def _():
            pltpu.async_copy(
                rows[p].at[pl.ds(0, TAIL)],
                out_hbm.at[b_g, pl.ds(s_off, TAIL)],
                wsem[p],
            )

    issue_gather(0, 0)
    step(0, 0, 1)

    def loop_body(it, carry):
        base = 1 + it * 3
        step(base, 1, 2)
        step(base + 1, 2, 0)
        step(base + 2, 0, 1)
        return carry

    lax.fori_loop(0, (N_ITER - 1) // 3, loop_body, 0)

    wait_write(2, N_CHUNK - 1)
    wait_write(0, N_CHUNK - 1)


@jax.jit
def _impl(x, table):
    xp = jnp.pad(x, ((0, 0), (0, SEQ_PAD - SEQ)))
    pe = jnp.asarray(_PE_CONST)
    mesh = plsc.VectorSubcoreMesh(core_axis_name="c", subcore_axis_name="s")
    k = pl.kernel(
        _sc_body,
        mesh=mesh,
        out_type=jax.ShapeDtypeStruct((BATCH, SEQ, D_DIM), jnp.float32),
        scratch_types=[
            pltpu.VMEM((B_PER_W, SEQ_PAD), jnp.int32),
            pltpu.VMEM((CHUNK, D_DIM), jnp.float32),
            pltpu.VMEM((CHUNK, D_DIM), jnp.float32),
            pltpu.VMEM((CHUNK, D_DIM), jnp.float32),
            pltpu.VMEM((CHUNK, D_DIM), jnp.float32),
            pltpu.SemaphoreType.DMA,
            pltpu.SemaphoreType.DMA,
            pltpu.SemaphoreType.DMA,
            pltpu.SemaphoreType.DMA,
            pltpu.SemaphoreType.DMA,
            pltpu.SemaphoreType.DMA,
        ],
        compiler_params=pltpu.CompilerParams(use_tc_tiling_on_sc=False),
    )
    return k(xp, table, pe)


def kernel(x, table):
    return _impl(x, table)
